# Initial kernel scaffold; baseline (speedup 1.0000x reference)
import jax
import jax.numpy as jnp
from jax.experimental import pallas as pl


def kernel(x, src, dst, edge_weights, bias):
    raise NotImplementedError("write your pallas kernel here")


# speedup vs baseline: 122.5397x; 122.5397x over previous
import functools

import jax
import jax.numpy as jnp
from jax import lax
from jax.experimental import pallas as pl
from jax.experimental.pallas import tpu as pltpu
from jax.experimental.pallas import tpu_sc as plsc

SIZE = 65536
MAX_EDGES = 4194304
NC = 2
NS = 16
NW = NC * NS
EPW = MAX_EDGES // NW
CHUNK = 4096
NCHUNK = EPW // CHUNK
L = 16

_mesh = plsc.VectorSubcoreMesh(core_axis_name="c", subcore_axis_name="s")


@functools.partial(
    pl.kernel,
    mesh=_mesh,
    out_type=jax.ShapeDtypeStruct((NW, SIZE), jnp.float32),
    scratch_types=[
        pltpu.VMEM((SIZE,), jnp.float32),
        pltpu.VMEM((CHUNK,), jnp.int32),
        pltpu.VMEM((CHUNK,), jnp.int32),
        pltpu.VMEM((CHUNK,), jnp.float32),
        pltpu.VMEM((CHUNK,), jnp.float32),
        pltpu.SemaphoreType.DMA,
    ],
    compiler_params=pltpu.CompilerParams(needs_layout_passes=False),
)
def _sc_scatter(x_hbm, src_hbm, dst_hbm, w_hbm, part_hbm,
                acc, src_v, dst_v, w_v, vals_v, sem):
    cid = lax.axis_index("c")
    sid = lax.axis_index("s")
    wid = sid * NC + cid
    base = wid * EPW

    zeros = jnp.zeros((L,), jnp.float32)

    def zero_body(i, carry):
        acc[pl.ds(i * L, L)] = zeros
        return carry

    lax.fori_loop(0, SIZE // L, zero_body, 0)

    def chunk_body(c, carry):
        off = base + c * CHUNK
        pltpu.sync_copy(src_hbm.at[pl.ds(off, CHUNK)], src_v)
        pltpu.sync_copy(dst_hbm.at[pl.ds(off, CHUNK)], dst_v)
        pltpu.sync_copy(w_hbm.at[pl.ds(off, CHUNK)], w_v)
        pltpu.async_copy(x_hbm.at[src_v], vals_v, sem).wait()

        def edge_body(j, inner):
            sl = pl.ds(j * L, L)
            v = vals_v[sl] * w_v[sl]
            plsc.addupdate_scatter(acc, [dst_v[sl]], v)
            return inner

        lax.fori_loop(0, CHUNK // L, edge_body, 0)
        return carry

    lax.fori_loop(0, NCHUNK, chunk_body, 0)
    pltpu.sync_copy(acc, part_hbm.at[wid])


def _combine_body(part_ref, bias_ref, out_ref):
    out_ref[:] = jnp.sum(part_ref[:], axis=0) + bias_ref[:]


@jax.jit
def kernel(x, src, dst, edge_weights, bias):
    partials = _sc_scatter(x, src, dst, edge_weights)
    out = pl.pallas_call(
        _combine_body,
        out_shape=jax.ShapeDtypeStruct((SIZE,), jnp.float32),
    )(partials, bias)
    return out
